# baseline (speedup 1.0000x reference)
import numpy as np
import jax
import jax.numpy as jnp
from jax.experimental import pallas as pl
from jax.experimental.pallas import tpu as pltpu

_HEAD_INDEX = [1, 2, 3, 4, 5, 6, 7, 8, 9, 10, 11, 12, 13, 17, 18]
_REGIONS = [[1], [17, 18], [4, 5, 6], [2, 3], [7, 8, 9], [10], [12, 13], [11]]
_TEMP = 0.01
_EPS = 1e-8
_NEG = -1e30

_LUT = np.full((19,), -1.0, np.float32)
for _r, _grp in enumerate(_REGIONS):
    for _l in _grp:
        _LUT[_l] = float(_r)

_T_S2D = np.zeros((3, 3, 2, 2, 2, 2, 3, 3), np.float32)
for _qy in range(3):
    for _qx in range(3):
        for _py in range(2):
            for _px in range(2):
                for _oy in range(2):
                    for _ox in range(2):
                        for _dy in range(3):
                            for _dx in range(3):
                                if (2 * _qy + _py == _oy + _dy + 1
                                        and 2 * _qx + _px == _ox + _dx + 1):
                                    _T_S2D[_qy, _qx, _py, _px, _oy, _ox, _dy, _dx] = 1.0


def _s2d_weights(w):
    co, ci = w.shape[0], w.shape[1]
    t = jnp.asarray(_T_S2D)
    ws = jnp.einsum('abcdefgh,migh->abcdiefm', t, w)
    return ws.reshape(9, 4 * ci, 4 * co)


_TH = 16


def _conv_s2d_tap_kernel(x_ref, w_ref, out_ref, *, cin, relu, transpose_out):
    t = pl.program_id(1)
    acc = None
    for k in range(9):
        qy, qx = k // 3, k % 3
        sl = x_ref[0, pl.ds(t * _TH + qy, _TH), qx:qx + 112, :]
        sl = sl.reshape(_TH * 112, cin)
        p = jax.lax.dot_general(
            sl, w_ref[k], (((1,), (0,)), ((), ())),
            precision=jax.lax.Precision.HIGHEST,
            preferred_element_type=jnp.float32)
        acc = p if acc is None else acc + p
    if relu:
        acc = jnp.maximum(acc, 0.0)
    if transpose_out:
        out_ref[0] = acc.T
    else:
        out_ref[0] = acc


def _conv_s2d(xpad, ws, relu, transpose_out):
    B = xpad.shape[0]
    cin, cout = ws.shape[1], ws.shape[2]
    nt = 112 // _TH
    if transpose_out:
        out_spec = pl.BlockSpec((1, cout, _TH * 112), lambda b, t: (b, 0, t))
        out_shape = jax.ShapeDtypeStruct((B, cout, 12544), jnp.float32)
    else:
        out_spec = pl.BlockSpec((1, _TH * 112, cout), lambda b, t: (b, t, 0))
        out_shape = jax.ShapeDtypeStruct((B, 12544, cout), jnp.float32)
    import functools
    return pl.pallas_call(
        functools.partial(_conv_s2d_tap_kernel, cin=cin, relu=relu,
                          transpose_out=transpose_out),
        grid=(B, nt),
        in_specs=[
            pl.BlockSpec((1, 114, 114, cin), lambda b, t: (b, 0, 0, 0)),
            pl.BlockSpec((9, cin, cout), lambda b, t: (0, 0, 0)),
        ],
        out_specs=out_spec,
        out_shape=out_shape,
    )(xpad, ws)


def _corr_kernel(fa_ref, ft_ref, itr_ref, rar_ref, rtr_ref, iar_ref, itm_ref,
                 genh_ref, geni_ref):
    fa = fa_ref[0]
    ft = ft_ref[0]
    itr = itr_ref[0]
    rtr = rtr_ref[0]
    itm = itm_ref[0]
    rac = jnp.transpose(rar_ref[0])
    iac = jnp.transpose(iar_ref[0])

    def _norm(x):
        x = x - jnp.mean(x, axis=0, keepdims=True)
        n = jnp.sqrt(jnp.sum(x * x, axis=0, keepdims=True)) + _EPS
        return x / n

    fan = _norm(fa)
    ftn = _norm(ft)
    logits = jax.lax.dot_general(
        fan, ftn, (((0,), (0,)), ((), ())),
        precision=jax.lax.Precision.HIGHEST,
        preferred_element_type=jnp.float32) * (1.0 / _TEMP)

    mh = jnp.logical_and(rac == rtr, rac >= 0.0)
    lh = jnp.where(mh, logits, _NEG)
    mxh = jnp.max(lh, axis=1, keepdims=True)
    ph = jnp.exp(lh - mxh)
    fh = ph / jnp.sum(ph, axis=1, keepdims=True)
    fh = jnp.where(mxh > 0.5 * _NEG, fh, 0.0)
    genh_ref[0] = jax.lax.dot_general(
        itr, fh, (((1,), (1,)), ((), ())),
        precision=jax.lax.Precision.HIGHEST,
        preferred_element_type=jnp.float32)

    li = jnp.where(itm > 0.5, logits, _NEG)
    mxi = jnp.max(li, axis=1, keepdims=True)
    pi = jnp.exp(li - mxi)
    fi = pi / jnp.sum(pi, axis=1, keepdims=True)
    keep = jnp.logical_and(iac > 0.5, mxi > 0.5 * _NEG)
    fi = jnp.where(keep, fi, 0.0)
    geni_ref[0] = jax.lax.dot_general(
        itr, fi, (((1,), (1,)), ((), ())),
        precision=jax.lax.Precision.HIGHEST,
        preferred_element_type=jnp.float32)


def _conv2d(x, w):
    return jax.lax.conv_general_dilated(
        x, w, (1, 1), 'SAME', dimension_numbers=('NCHW', 'OIHW', 'NCHW'))


def _maxpool2(x):
    return jax.lax.reduce_window(x, -jnp.inf, jax.lax.max,
                                 (1, 1, 2, 2), (1, 1, 2, 2), 'VALID')


def _dilate(m, k=3):
    p = k // 2
    return jax.lax.reduce_window(m.astype(jnp.float32), -jnp.inf, jax.lax.max,
                                 (1, 1, k, k), (1, 1, 1, 1),
                                 [(0, 0), (0, 0), (p, p), (p, p)])


def kernel(I_a, I_gray, I_t, M_a, M_t, gt, Wf1, Wf2, Wf3, Wphi, Wth, Wd1, Wd2):
    B, _, H, W = I_a.shape

    x = jnp.concatenate([I_a, I_t], axis=0)
    x = _maxpool2(jax.nn.relu(_conv2d(x, Wf1)))
    x = _maxpool2(jax.nn.relu(_conv2d(x, Wf2)))
    x = _maxpool2(jax.nn.relu(_conv2d(x, Wf3)))
    fA = _conv2d(x[:B], Wphi)
    fT = _conv2d(x[B:], Wth)
    h, w = fA.shape[2], fA.shape[3]
    hw = h * w
    r = H // h

    head = jnp.asarray(_HEAD_INDEX)
    M_Ah = jnp.isin(M_a, head).astype(jnp.float32)
    M_Th = jnp.isin(M_t, head).astype(jnp.float32)
    M_Th_c = jnp.clip(M_Th, 0, 1)
    M_Ti = _dilate(M_Th_c) - M_Th_c
    s = jnp.clip(M_Ah + M_Th, 0, 1)
    M_Ad = _dilate(s)
    M_Ai = M_Ad - M_Ah

    def _region_id(lbl):
        rid = jnp.full(lbl.shape, -1.0, jnp.float32)
        for ridx, grp in enumerate(_REGIONS):
            hit = lbl == grp[0]
            for g in grp[1:]:
                hit = jnp.logical_or(hit, lbl == g)
            rid = jnp.where(hit, float(ridx), rid)
        return rid

    ra = _region_id(M_a[:, 0, ::r, ::r]).reshape(B, hw)
    rt = _region_id(M_t[:, 0, ::r, ::r]).reshape(B, hw)
    ia = M_Ai[:, 0, ::r, ::r].reshape(B, hw)
    it = M_Ti[:, 0, ::r, ::r].reshape(B, hw)

    itr = I_t.reshape(B, 3, h, r, w, r).mean(axis=(3, 5)).reshape(B, 3, hw)

    C = fA.shape[1]
    genh, geni = pl.pallas_call(
        _corr_kernel,
        grid=(B,),
        in_specs=[
            pl.BlockSpec((1, C, hw), lambda b: (b, 0, 0)),
            pl.BlockSpec((1, C, hw), lambda b: (b, 0, 0)),
            pl.BlockSpec((1, 3, hw), lambda b: (b, 0, 0)),
            pl.BlockSpec((1, 1, hw), lambda b: (b, 0, 0)),
            pl.BlockSpec((1, 1, hw), lambda b: (b, 0, 0)),
            pl.BlockSpec((1, 1, hw), lambda b: (b, 0, 0)),
            pl.BlockSpec((1, 1, hw), lambda b: (b, 0, 0)),
        ],
        out_specs=[
            pl.BlockSpec((1, 3, hw), lambda b: (b, 0, 0)),
            pl.BlockSpec((1, 3, hw), lambda b: (b, 0, 0)),
        ],
        out_shape=[
            jax.ShapeDtypeStruct((B, 3, hw), jnp.float32),
            jax.ShapeDtypeStruct((B, 3, hw), jnp.float32),
        ],
    )(fA.reshape(B, C, hw), fT.reshape(B, C, hw), itr,
      ra.reshape(B, 1, hw), rt.reshape(B, 1, hw),
      ia.reshape(B, 1, hw), it.reshape(B, 1, hw))

    gen_h = jnp.repeat(jnp.repeat(genh.reshape(B, 3, h, w), r, axis=2), r, axis=3)
    gen_i = jnp.repeat(jnp.repeat(geni.reshape(B, 3, h, w), r, axis=2), r, axis=3)

    I_tb = gt * (1.0 - M_Ad)
    I_ag = I_gray * M_Ah
    inp = jnp.concatenate([gen_h, gen_i, M_Ah, I_tb, M_Ai, I_ag], axis=1)

    xs = inp.reshape(B, 12, 112, 2, 112, 2).transpose(0, 2, 4, 3, 5, 1)
    xs = xs.reshape(B, 112, 112, 48)
    xs = jnp.pad(xs, ((0, 0), (1, 1), (1, 1), (0, 0)))
    w1s = _s2d_weights(Wd1)
    w2s = _s2d_weights(Wd2)
    y1 = _conv_s2d(xs, w1s, relu=True, transpose_out=False)
    y1 = jnp.pad(y1.reshape(B, 112, 112, 256),
                 ((0, 0), (1, 1), (1, 1), (0, 0)))
    y2 = _conv_s2d(y1, w2s, relu=False, transpose_out=True)
    oup = y2.reshape(B, 2, 2, 3, 112, 112).transpose(0, 3, 4, 1, 5, 2)
    return oup.reshape(B, 3, 224, 224)


# speedup vs baseline: 1.3000x; 1.3000x over previous
import numpy as np
import jax
import jax.numpy as jnp
from jax.experimental import pallas as pl
from jax.experimental.pallas import tpu as pltpu

_HEAD_INDEX = [1, 2, 3, 4, 5, 6, 7, 8, 9, 10, 11, 12, 13, 17, 18]
_REGIONS = [[1], [17, 18], [4, 5, 6], [2, 3], [7, 8, 9], [10], [12, 13], [11]]
_TEMP = 0.01
_EPS = 1e-8
_NEG = -1e30

_LUT = np.full((19,), -1.0, np.float32)
for _r, _grp in enumerate(_REGIONS):
    for _l in _grp:
        _LUT[_l] = float(_r)

_T_S2D = np.zeros((3, 3, 2, 2, 2, 2, 3, 3), np.float32)
for _qy in range(3):
    for _qx in range(3):
        for _py in range(2):
            for _px in range(2):
                for _oy in range(2):
                    for _ox in range(2):
                        for _dy in range(3):
                            for _dx in range(3):
                                if (2 * _qy + _py == _oy + _dy + 1
                                        and 2 * _qx + _px == _ox + _dx + 1):
                                    _T_S2D[_qy, _qx, _py, _px, _oy, _ox, _dy, _dx] = 1.0


def _s2d_weights(w):
    co, ci = w.shape[0], w.shape[1]
    t = jnp.asarray(_T_S2D)
    ws = jnp.einsum('abcdefgh,migh->abcdiefm', t, w)
    return ws.reshape(9, 4 * ci, 4 * co)


_TH = 16


def _conv_s2d_tap_kernel(x_ref, w_ref, out_ref, *, cin, relu, transpose_out):
    t = pl.program_id(1)
    acc = None
    for k in range(9):
        qy, qx = k // 3, k % 3
        sl = x_ref[0, pl.ds(t * _TH + qy, _TH), qx:qx + 112, :]
        sl = sl.reshape(_TH * 112, cin)
        p = jax.lax.dot_general(
            sl, w_ref[k], (((1,), (0,)), ((), ())),
            preferred_element_type=jnp.float32)
        acc = p if acc is None else acc + p
    if relu:
        acc = jnp.maximum(acc, 0.0)
    if transpose_out:
        out_ref[0] = acc.T
    else:
        out_ref[0] = acc


def _conv_s2d(xpad, ws, relu, transpose_out):
    B = xpad.shape[0]
    cin, cout = ws.shape[1], ws.shape[2]
    nt = 112 // _TH
    if transpose_out:
        out_spec = pl.BlockSpec((1, cout, _TH * 112), lambda b, t: (b, 0, t))
        out_shape = jax.ShapeDtypeStruct((B, cout, 12544), jnp.float32)
    else:
        out_spec = pl.BlockSpec((1, _TH * 112, cout), lambda b, t: (b, t, 0))
        out_shape = jax.ShapeDtypeStruct((B, 12544, cout), jnp.float32)
    import functools
    return pl.pallas_call(
        functools.partial(_conv_s2d_tap_kernel, cin=cin, relu=relu,
                          transpose_out=transpose_out),
        grid=(B, nt),
        in_specs=[
            pl.BlockSpec((1, 114, 114, cin), lambda b, t: (b, 0, 0, 0)),
            pl.BlockSpec((9, cin, cout), lambda b, t: (0, 0, 0)),
        ],
        out_specs=out_spec,
        out_shape=out_shape,
    )(xpad, ws)


def _corr_kernel(fa_ref, ft_ref, itr_ref, rar_ref, rtr_ref, iar_ref, itm_ref,
                 genh_ref, geni_ref):
    fa = fa_ref[0]
    ft = ft_ref[0]
    itr = itr_ref[0]
    rtr = rtr_ref[0]
    itm = itm_ref[0]
    rac = jnp.transpose(rar_ref[0])
    iac = jnp.transpose(iar_ref[0])

    def _norm(x):
        x = x - jnp.mean(x, axis=0, keepdims=True)
        n = jnp.sqrt(jnp.sum(x * x, axis=0, keepdims=True)) + _EPS
        return x / n

    fan = _norm(fa)
    ftn = _norm(ft)
    logits = jax.lax.dot_general(
        fan, ftn, (((0,), (0,)), ((), ())),
        precision=jax.lax.Precision.HIGHEST,
        preferred_element_type=jnp.float32) * (1.0 / _TEMP)

    mh = jnp.logical_and(rac == rtr, rac >= 0.0)
    lh = jnp.where(mh, logits, _NEG)
    mxh = jnp.max(lh, axis=1, keepdims=True)
    ph = jnp.exp(lh - mxh)
    fh = ph / jnp.sum(ph, axis=1, keepdims=True)
    fh = jnp.where(mxh > 0.5 * _NEG, fh, 0.0)
    genh_ref[0] = jax.lax.dot_general(
        itr, fh, (((1,), (1,)), ((), ())),
        preferred_element_type=jnp.float32)

    li = jnp.where(itm > 0.5, logits, _NEG)
    mxi = jnp.max(li, axis=1, keepdims=True)
    pi = jnp.exp(li - mxi)
    fi = pi / jnp.sum(pi, axis=1, keepdims=True)
    keep = jnp.logical_and(iac > 0.5, mxi > 0.5 * _NEG)
    fi = jnp.where(keep, fi, 0.0)
    geni_ref[0] = jax.lax.dot_general(
        itr, fi, (((1,), (1,)), ((), ())),
        preferred_element_type=jnp.float32)


def _conv2d(x, w):
    return jax.lax.conv_general_dilated(
        x, w, (1, 1), 'SAME', dimension_numbers=('NCHW', 'OIHW', 'NCHW'))


def _maxpool2(x):
    return jax.lax.reduce_window(x, -jnp.inf, jax.lax.max,
                                 (1, 1, 2, 2), (1, 1, 2, 2), 'VALID')


def _dilate(m, k=3):
    p = k // 2
    return jax.lax.reduce_window(m.astype(jnp.float32), -jnp.inf, jax.lax.max,
                                 (1, 1, k, k), (1, 1, 1, 1),
                                 [(0, 0), (0, 0), (p, p), (p, p)])


def kernel(I_a, I_gray, I_t, M_a, M_t, gt, Wf1, Wf2, Wf3, Wphi, Wth, Wd1, Wd2):
    B, _, H, W = I_a.shape

    x = jnp.concatenate([I_a, I_t], axis=0)
    x = _maxpool2(jax.nn.relu(_conv2d(x, Wf1)))
    x = _maxpool2(jax.nn.relu(_conv2d(x, Wf2)))
    x = _maxpool2(jax.nn.relu(_conv2d(x, Wf3)))
    fA = _conv2d(x[:B], Wphi)
    fT = _conv2d(x[B:], Wth)
    h, w = fA.shape[2], fA.shape[3]
    hw = h * w
    r = H // h

    head = jnp.asarray(_HEAD_INDEX)
    M_Ah = jnp.isin(M_a, head).astype(jnp.float32)
    M_Th = jnp.isin(M_t, head).astype(jnp.float32)
    M_Th_c = jnp.clip(M_Th, 0, 1)
    M_Ti = _dilate(M_Th_c) - M_Th_c
    s = jnp.clip(M_Ah + M_Th, 0, 1)
    M_Ad = _dilate(s)
    M_Ai = M_Ad - M_Ah

    def _region_id(lbl):
        rid = jnp.full(lbl.shape, -1.0, jnp.float32)
        for ridx, grp in enumerate(_REGIONS):
            hit = lbl == grp[0]
            for g in grp[1:]:
                hit = jnp.logical_or(hit, lbl == g)
            rid = jnp.where(hit, float(ridx), rid)
        return rid

    ra = _region_id(M_a[:, 0, ::r, ::r]).reshape(B, hw)
    rt = _region_id(M_t[:, 0, ::r, ::r]).reshape(B, hw)
    ia = M_Ai[:, 0, ::r, ::r].reshape(B, hw)
    it = M_Ti[:, 0, ::r, ::r].reshape(B, hw)

    itr = I_t.reshape(B, 3, h, r, w, r).mean(axis=(3, 5)).reshape(B, 3, hw)

    C = fA.shape[1]
    genh, geni = pl.pallas_call(
        _corr_kernel,
        grid=(B,),
        in_specs=[
            pl.BlockSpec((1, C, hw), lambda b: (b, 0, 0)),
            pl.BlockSpec((1, C, hw), lambda b: (b, 0, 0)),
            pl.BlockSpec((1, 3, hw), lambda b: (b, 0, 0)),
            pl.BlockSpec((1, 1, hw), lambda b: (b, 0, 0)),
            pl.BlockSpec((1, 1, hw), lambda b: (b, 0, 0)),
            pl.BlockSpec((1, 1, hw), lambda b: (b, 0, 0)),
            pl.BlockSpec((1, 1, hw), lambda b: (b, 0, 0)),
        ],
        out_specs=[
            pl.BlockSpec((1, 3, hw), lambda b: (b, 0, 0)),
            pl.BlockSpec((1, 3, hw), lambda b: (b, 0, 0)),
        ],
        out_shape=[
            jax.ShapeDtypeStruct((B, 3, hw), jnp.float32),
            jax.ShapeDtypeStruct((B, 3, hw), jnp.float32),
        ],
    )(fA.reshape(B, C, hw), fT.reshape(B, C, hw), itr,
      ra.reshape(B, 1, hw), rt.reshape(B, 1, hw),
      ia.reshape(B, 1, hw), it.reshape(B, 1, hw))

    gen_h = jnp.repeat(jnp.repeat(genh.reshape(B, 3, h, w), r, axis=2), r, axis=3)
    gen_i = jnp.repeat(jnp.repeat(geni.reshape(B, 3, h, w), r, axis=2), r, axis=3)

    I_tb = gt * (1.0 - M_Ad)
    I_ag = I_gray * M_Ah
    inp = jnp.concatenate([gen_h, gen_i, M_Ah, I_tb, M_Ai, I_ag], axis=1)

    xs = inp.reshape(B, 12, 112, 2, 112, 2).transpose(0, 2, 4, 3, 5, 1)
    xs = xs.reshape(B, 112, 112, 48)
    xs = jnp.pad(xs, ((0, 0), (1, 1), (1, 1), (0, 0)))
    w1s = _s2d_weights(Wd1)
    w2s = _s2d_weights(Wd2)
    y1 = _conv_s2d(xs, w1s, relu=True, transpose_out=False)
    y1 = jnp.pad(y1.reshape(B, 112, 112, 256),
                 ((0, 0), (1, 1), (1, 1), (0, 0)))
    y2 = _conv_s2d(y1, w2s, relu=False, transpose_out=True)
    oup = y2.reshape(B, 2, 2, 3, 112, 112).transpose(0, 3, 4, 1, 5, 2)
    return oup.reshape(B, 3, 224, 224)


# speedup vs baseline: 1.3105x; 1.0080x over previous
import numpy as np
import jax
import jax.numpy as jnp
from jax.experimental import pallas as pl
from jax.experimental.pallas import tpu as pltpu

_HEAD_INDEX = [1, 2, 3, 4, 5, 6, 7, 8, 9, 10, 11, 12, 13, 17, 18]
_REGIONS = [[1], [17, 18], [4, 5, 6], [2, 3], [7, 8, 9], [10], [12, 13], [11]]
_TEMP = 0.01
_EPS = 1e-8
_NEG = -1e30

_LUT = np.full((19,), -1.0, np.float32)
for _r, _grp in enumerate(_REGIONS):
    for _l in _grp:
        _LUT[_l] = float(_r)

_T_S2D = np.zeros((3, 3, 2, 2, 2, 2, 3, 3), np.float32)
for _qy in range(3):
    for _qx in range(3):
        for _py in range(2):
            for _px in range(2):
                for _oy in range(2):
                    for _ox in range(2):
                        for _dy in range(3):
                            for _dx in range(3):
                                if (2 * _qy + _py == _oy + _dy + 1
                                        and 2 * _qx + _px == _ox + _dx + 1):
                                    _T_S2D[_qy, _qx, _py, _px, _oy, _ox, _dy, _dx] = 1.0


def _s2d_weights(w):
    co, ci = w.shape[0], w.shape[1]
    t = jnp.asarray(_T_S2D)
    ws = jnp.einsum('abcdefgh,migh->abcdiefm', t, w)
    return ws.reshape(9, 4 * ci, 4 * co)


_TH = 16


def _decoder_kernel(x_ref, w1_ref, w2_ref, out_ref, xp2_ref):
    xp2_ref[...] = jnp.zeros(xp2_ref.shape, jnp.float32)
    for t in range(112 // _TH):
        acc = None
        for k in range(9):
            qy, qx = k // 3, k % 3
            sl = x_ref[0, t * _TH + qy:t * _TH + qy + _TH, qx:qx + 112, :]
            p = jax.lax.dot_general(
                sl.reshape(_TH * 112, 48), w1_ref[k], (((1,), (0,)), ((), ())),
                preferred_element_type=jnp.float32)
            acc = p if acc is None else acc + p
        xp2_ref[1 + t * _TH:1 + (t + 1) * _TH, 1:113, :] = (
            jnp.maximum(acc, 0.0).reshape(_TH, 112, 256))
    for t in range(112 // _TH):
        acc = None
        for k in range(9):
            qy, qx = k // 3, k % 3
            sl = xp2_ref[t * _TH + qy:t * _TH + qy + _TH, qx:qx + 112, :]
            p = jax.lax.dot_general(
                sl.reshape(_TH * 112, 256), w2_ref[k], (((1,), (0,)), ((), ())),
                preferred_element_type=jnp.float32)
            acc = p if acc is None else acc + p
        out_ref[0, :, t * _TH * 112:(t + 1) * _TH * 112] = acc.T


def _decoder(xpad, w1s, w2s):
    B = xpad.shape[0]
    return pl.pallas_call(
        _decoder_kernel,
        grid=(B,),
        in_specs=[
            pl.BlockSpec((1, 114, 114, 48), lambda b: (b, 0, 0, 0)),
            pl.BlockSpec((9, 48, 256), lambda b: (0, 0, 0)),
            pl.BlockSpec((9, 256, 12), lambda b: (0, 0, 0)),
        ],
        out_specs=pl.BlockSpec((1, 12, 12544), lambda b: (b, 0, 0)),
        out_shape=jax.ShapeDtypeStruct((B, 12, 12544), jnp.float32),
        scratch_shapes=[pltpu.VMEM((114, 114, 256), jnp.float32)],
    )(xpad, w1s, w2s)


def _corr_kernel(fa_ref, ft_ref, itr_ref, rar_ref, rtr_ref, iar_ref, itm_ref,
                 genh_ref, geni_ref):
    fa = fa_ref[0]
    ft = ft_ref[0]
    itr = itr_ref[0]
    rtr = rtr_ref[0]
    itm = itm_ref[0]
    rac = jnp.transpose(rar_ref[0])
    iac = jnp.transpose(iar_ref[0])

    def _norm(x):
        x = x - jnp.mean(x, axis=0, keepdims=True)
        n = jnp.sqrt(jnp.sum(x * x, axis=0, keepdims=True)) + _EPS
        return x / n

    fan = _norm(fa)
    ftn = _norm(ft)
    logits = jax.lax.dot_general(
        fan, ftn, (((0,), (0,)), ((), ())),
        precision=jax.lax.Precision.HIGHEST,
        preferred_element_type=jnp.float32) * (1.0 / _TEMP)

    mh = jnp.logical_and(rac == rtr, rac >= 0.0)
    lh = jnp.where(mh, logits, _NEG)
    mxh = jnp.max(lh, axis=1, keepdims=True)
    ph = jnp.exp(lh - mxh)
    fh = ph / jnp.sum(ph, axis=1, keepdims=True)
    fh = jnp.where(mxh > 0.5 * _NEG, fh, 0.0)
    genh_ref[0] = jax.lax.dot_general(
        itr, fh, (((1,), (1,)), ((), ())),
        preferred_element_type=jnp.float32)

    li = jnp.where(itm > 0.5, logits, _NEG)
    mxi = jnp.max(li, axis=1, keepdims=True)
    pi = jnp.exp(li - mxi)
    fi = pi / jnp.sum(pi, axis=1, keepdims=True)
    keep = jnp.logical_and(iac > 0.5, mxi > 0.5 * _NEG)
    fi = jnp.where(keep, fi, 0.0)
    geni_ref[0] = jax.lax.dot_general(
        itr, fi, (((1,), (1,)), ((), ())),
        preferred_element_type=jnp.float32)


def _conv2d(x, w):
    return jax.lax.conv_general_dilated(
        x, w, (1, 1), 'SAME', dimension_numbers=('NCHW', 'OIHW', 'NCHW'))


def _maxpool2(x):
    return jax.lax.reduce_window(x, -jnp.inf, jax.lax.max,
                                 (1, 1, 2, 2), (1, 1, 2, 2), 'VALID')


def _dilate(m, k=3):
    p = k // 2
    return jax.lax.reduce_window(m.astype(jnp.float32), -jnp.inf, jax.lax.max,
                                 (1, 1, k, k), (1, 1, 1, 1),
                                 [(0, 0), (0, 0), (p, p), (p, p)])


def kernel(I_a, I_gray, I_t, M_a, M_t, gt, Wf1, Wf2, Wf3, Wphi, Wth, Wd1, Wd2):
    B, _, H, W = I_a.shape

    x = jnp.concatenate([I_a, I_t], axis=0)
    x = _maxpool2(jax.nn.relu(_conv2d(x, Wf1)))
    x = _maxpool2(jax.nn.relu(_conv2d(x, Wf2)))
    x = _maxpool2(jax.nn.relu(_conv2d(x, Wf3)))
    fA = _conv2d(x[:B], Wphi)
    fT = _conv2d(x[B:], Wth)
    h, w = fA.shape[2], fA.shape[3]
    hw = h * w
    r = H // h

    head = jnp.asarray(_HEAD_INDEX)
    M_Ah = jnp.isin(M_a, head).astype(jnp.float32)
    M_Th = jnp.isin(M_t, head).astype(jnp.float32)
    M_Th_c = jnp.clip(M_Th, 0, 1)
    M_Ti = _dilate(M_Th_c) - M_Th_c
    s = jnp.clip(M_Ah + M_Th, 0, 1)
    M_Ad = _dilate(s)
    M_Ai = M_Ad - M_Ah

    def _region_id(lbl):
        rid = jnp.full(lbl.shape, -1.0, jnp.float32)
        for ridx, grp in enumerate(_REGIONS):
            hit = lbl == grp[0]
            for g in grp[1:]:
                hit = jnp.logical_or(hit, lbl == g)
            rid = jnp.where(hit, float(ridx), rid)
        return rid

    ra = _region_id(M_a[:, 0, ::r, ::r]).reshape(B, hw)
    rt = _region_id(M_t[:, 0, ::r, ::r]).reshape(B, hw)
    ia = M_Ai[:, 0, ::r, ::r].reshape(B, hw)
    it = M_Ti[:, 0, ::r, ::r].reshape(B, hw)

    itr = I_t.reshape(B, 3, h, r, w, r).mean(axis=(3, 5)).reshape(B, 3, hw)

    C = fA.shape[1]
    genh, geni = pl.pallas_call(
        _corr_kernel,
        grid=(B,),
        in_specs=[
            pl.BlockSpec((1, C, hw), lambda b: (b, 0, 0)),
            pl.BlockSpec((1, C, hw), lambda b: (b, 0, 0)),
            pl.BlockSpec((1, 3, hw), lambda b: (b, 0, 0)),
            pl.BlockSpec((1, 1, hw), lambda b: (b, 0, 0)),
            pl.BlockSpec((1, 1, hw), lambda b: (b, 0, 0)),
            pl.BlockSpec((1, 1, hw), lambda b: (b, 0, 0)),
            pl.BlockSpec((1, 1, hw), lambda b: (b, 0, 0)),
        ],
        out_specs=[
            pl.BlockSpec((1, 3, hw), lambda b: (b, 0, 0)),
            pl.BlockSpec((1, 3, hw), lambda b: (b, 0, 0)),
        ],
        out_shape=[
            jax.ShapeDtypeStruct((B, 3, hw), jnp.float32),
            jax.ShapeDtypeStruct((B, 3, hw), jnp.float32),
        ],
    )(fA.reshape(B, C, hw), fT.reshape(B, C, hw), itr,
      ra.reshape(B, 1, hw), rt.reshape(B, 1, hw),
      ia.reshape(B, 1, hw), it.reshape(B, 1, hw))

    gen_h = jnp.repeat(jnp.repeat(genh.reshape(B, 3, h, w), r, axis=2), r, axis=3)
    gen_i = jnp.repeat(jnp.repeat(geni.reshape(B, 3, h, w), r, axis=2), r, axis=3)

    I_tb = gt * (1.0 - M_Ad)
    I_ag = I_gray * M_Ah
    inp = jnp.concatenate([gen_h, gen_i, M_Ah, I_tb, M_Ai, I_ag], axis=1)

    xs = inp.reshape(B, 12, 112, 2, 112, 2).transpose(0, 2, 4, 3, 5, 1)
    xs = xs.reshape(B, 112, 112, 48)
    xs = jnp.pad(xs, ((0, 0), (1, 1), (1, 1), (0, 0)))
    w1s = _s2d_weights(Wd1)
    w2s = _s2d_weights(Wd2)
    y2 = _decoder(xs, w1s, w2s)
    oup = y2.reshape(B, 2, 2, 3, 112, 112).transpose(0, 3, 4, 1, 5, 2)
    return oup.reshape(B, 3, 224, 224)


# speedup vs baseline: 2.5740x; 1.9642x over previous
import numpy as np
import jax
import jax.numpy as jnp
from jax.experimental import pallas as pl
from jax.experimental.pallas import tpu as pltpu

_HEAD_INDEX = [1, 2, 3, 4, 5, 6, 7, 8, 9, 10, 11, 12, 13, 17, 18]
_REGIONS = [[1], [17, 18], [4, 5, 6], [2, 3], [7, 8, 9], [10], [12, 13], [11]]
_TEMP = 0.01
_EPS = 1e-8
_NEG = -1e30

_LUT = np.full((19,), -1.0, np.float32)
for _r, _grp in enumerate(_REGIONS):
    for _l in _grp:
        _LUT[_l] = float(_r)


_L1 = 114 * 256
_L2 = 112 * 256


def _dec_kernel(x_ref, w1_ref, w2_ref, out_ref, f2_ref):
    acc1 = None
    for k in range(9):
        dy, dx = k // 3, k % 3
        st = 127 + dy * 256 + dx
        sl = x_ref[0, 0, :, st:st + _L1]
        p = jax.lax.dot_general(
            w1_ref[k], sl, (((1,), (0,)), ((), ())),
            preferred_element_type=jnp.float32)
        acc1 = p if acc1 is None else acc1 + p
    t = pl.program_id(1)
    lane = jax.lax.broadcasted_iota(jnp.int32, (64, _L1), 1)
    lo = jnp.where(t == 0, 256, 0)
    hi = jnp.where(t == pl.num_programs(1) - 1, 113 * 256, _L1)
    ok = ((lane % 256) < 224) & (lane >= lo) & (lane < hi)
    acc1 = jnp.where(ok, jnp.maximum(acc1, 0.0), 0.0)
    f2_ref[...] = jnp.zeros(f2_ref.shape, jnp.float32)
    f2_ref[:, 128:128 + _L1] = acc1
    acc2 = None
    for k in range(9):
        dy, dx = k // 3, k % 3
        st = 127 + dy * 256 + dx
        sl = f2_ref[:, st:st + _L2]
        p = jax.lax.dot_general(
            w2_ref[k], sl, (((1,), (0,)), ((), ())),
            preferred_element_type=jnp.float32)
        acc2 = p if acc2 is None else acc2 + p
    out_ref[0, 0] = acc2


def _decoder(inp, Wd1, Wd2):
    B = inp.shape[0]
    xp = jnp.pad(inp, ((0, 0), (0, 0), (2, 2), (0, 32)))
    halves = jnp.stack([xp[:, :, 0:116], xp[:, :, 112:228]], axis=1)
    halves = halves.reshape(B, 2, 12, 116 * 256)
    halves = jnp.pad(halves, ((0, 0), (0, 0), (0, 0), (128, 128)))
    w1 = Wd1.transpose(2, 3, 0, 1).reshape(9, 64, 12)
    w2 = Wd2.transpose(2, 3, 0, 1).reshape(9, 3, 64)
    y = pl.pallas_call(
        _dec_kernel,
        grid=(B, 2),
        in_specs=[
            pl.BlockSpec((1, 1, 12, 29952), lambda b, t: (b, t, 0, 0)),
            pl.BlockSpec((9, 64, 12), lambda b, t: (0, 0, 0)),
            pl.BlockSpec((9, 3, 64), lambda b, t: (0, 0, 0)),
        ],
        out_specs=pl.BlockSpec((1, 1, 3, _L2), lambda b, t: (b, t, 0, 0)),
        out_shape=jax.ShapeDtypeStruct((B, 2, 3, _L2), jnp.float32),
        scratch_shapes=[pltpu.VMEM((64, 29440), jnp.float32)],
    )(halves, w1, w2)
    y = y.reshape(B, 2, 3, 112, 256)[:, :, :, :, :224]
    return y.transpose(0, 2, 1, 3, 4).reshape(B, 3, 224, 224)


def _corr_kernel(fa_ref, ft_ref, itr_ref, rar_ref, rtr_ref, iar_ref, itm_ref,
                 genh_ref, geni_ref):
    fa = fa_ref[0]
    ft = ft_ref[0]
    itr = itr_ref[0]
    rtr = rtr_ref[0]
    itm = itm_ref[0]
    rac = jnp.transpose(rar_ref[0])
    iac = jnp.transpose(iar_ref[0])

    def _norm(x):
        x = x - jnp.mean(x, axis=0, keepdims=True)
        n = jnp.sqrt(jnp.sum(x * x, axis=0, keepdims=True)) + _EPS
        return x / n

    fan = _norm(fa)
    ftn = _norm(ft)
    logits = jax.lax.dot_general(
        fan, ftn, (((0,), (0,)), ((), ())),
        precision=jax.lax.Precision.HIGHEST,
        preferred_element_type=jnp.float32) * (1.0 / _TEMP)

    mh = jnp.logical_and(rac == rtr, rac >= 0.0)
    lh = jnp.where(mh, logits, _NEG)
    mxh = jnp.max(lh, axis=1, keepdims=True)
    ph = jnp.exp(lh - mxh)
    fh = ph / jnp.sum(ph, axis=1, keepdims=True)
    fh = jnp.where(mxh > 0.5 * _NEG, fh, 0.0)
    genh_ref[0] = jax.lax.dot_general(
        itr, fh, (((1,), (1,)), ((), ())),
        precision=jax.lax.Precision.HIGHEST,
        preferred_element_type=jnp.float32)

    li = jnp.where(itm > 0.5, logits, _NEG)
    mxi = jnp.max(li, axis=1, keepdims=True)
    pi = jnp.exp(li - mxi)
    fi = pi / jnp.sum(pi, axis=1, keepdims=True)
    keep = jnp.logical_and(iac > 0.5, mxi > 0.5 * _NEG)
    fi = jnp.where(keep, fi, 0.0)
    geni_ref[0] = jax.lax.dot_general(
        itr, fi, (((1,), (1,)), ((), ())),
        precision=jax.lax.Precision.HIGHEST,
        preferred_element_type=jnp.float32)


def _conv2d(x, w):
    return jax.lax.conv_general_dilated(
        x, w, (1, 1), 'SAME', dimension_numbers=('NCHW', 'OIHW', 'NCHW'))


def _maxpool2(x):
    return jax.lax.reduce_window(x, -jnp.inf, jax.lax.max,
                                 (1, 1, 2, 2), (1, 1, 2, 2), 'VALID')


def _dilate(m, k=3):
    p = k // 2
    return jax.lax.reduce_window(m.astype(jnp.float32), -jnp.inf, jax.lax.max,
                                 (1, 1, k, k), (1, 1, 1, 1),
                                 [(0, 0), (0, 0), (p, p), (p, p)])


def kernel(I_a, I_gray, I_t, M_a, M_t, gt, Wf1, Wf2, Wf3, Wphi, Wth, Wd1, Wd2):
    B, _, H, W = I_a.shape

    x = jnp.concatenate([I_a, I_t], axis=0)
    x = _maxpool2(jax.nn.relu(_conv2d(x, Wf1)))
    x = _maxpool2(jax.nn.relu(_conv2d(x, Wf2)))
    x = _maxpool2(jax.nn.relu(_conv2d(x, Wf3)))
    fA = _conv2d(x[:B], Wphi)
    fT = _conv2d(x[B:], Wth)
    h, w = fA.shape[2], fA.shape[3]
    hw = h * w
    r = H // h

    head = jnp.asarray(_HEAD_INDEX)
    M_Ah = jnp.isin(M_a, head).astype(jnp.float32)
    M_Th = jnp.isin(M_t, head).astype(jnp.float32)
    M_Th_c = jnp.clip(M_Th, 0, 1)
    M_Ti = _dilate(M_Th_c) - M_Th_c
    s = jnp.clip(M_Ah + M_Th, 0, 1)
    M_Ad = _dilate(s)
    M_Ai = M_Ad - M_Ah

    def _region_id(lbl):
        rid = jnp.full(lbl.shape, -1.0, jnp.float32)
        for ridx, grp in enumerate(_REGIONS):
            hit = lbl == grp[0]
            for g in grp[1:]:
                hit = jnp.logical_or(hit, lbl == g)
            rid = jnp.where(hit, float(ridx), rid)
        return rid

    ra = _region_id(M_a[:, 0, ::r, ::r]).reshape(B, hw)
    rt = _region_id(M_t[:, 0, ::r, ::r]).reshape(B, hw)
    ia = M_Ai[:, 0, ::r, ::r].reshape(B, hw)
    it = M_Ti[:, 0, ::r, ::r].reshape(B, hw)

    itr = I_t.reshape(B, 3, h, r, w, r).mean(axis=(3, 5)).reshape(B, 3, hw)

    C = fA.shape[1]
    genh, geni = pl.pallas_call(
        _corr_kernel,
        grid=(B,),
        in_specs=[
            pl.BlockSpec((1, C, hw), lambda b: (b, 0, 0)),
            pl.BlockSpec((1, C, hw), lambda b: (b, 0, 0)),
            pl.BlockSpec((1, 3, hw), lambda b: (b, 0, 0)),
            pl.BlockSpec((1, 1, hw), lambda b: (b, 0, 0)),
            pl.BlockSpec((1, 1, hw), lambda b: (b, 0, 0)),
            pl.BlockSpec((1, 1, hw), lambda b: (b, 0, 0)),
            pl.BlockSpec((1, 1, hw), lambda b: (b, 0, 0)),
        ],
        out_specs=[
            pl.BlockSpec((1, 3, hw), lambda b: (b, 0, 0)),
            pl.BlockSpec((1, 3, hw), lambda b: (b, 0, 0)),
        ],
        out_shape=[
            jax.ShapeDtypeStruct((B, 3, hw), jnp.float32),
            jax.ShapeDtypeStruct((B, 3, hw), jnp.float32),
        ],
    )(fA.reshape(B, C, hw), fT.reshape(B, C, hw), itr,
      ra.reshape(B, 1, hw), rt.reshape(B, 1, hw),
      ia.reshape(B, 1, hw), it.reshape(B, 1, hw))

    gen_h = jnp.repeat(jnp.repeat(genh.reshape(B, 3, h, w), r, axis=2), r, axis=3)
    gen_i = jnp.repeat(jnp.repeat(geni.reshape(B, 3, h, w), r, axis=2), r, axis=3)

    I_tb = gt * (1.0 - M_Ad)
    I_ag = I_gray * M_Ah
    inp = jnp.concatenate([gen_h, gen_i, M_Ah, I_tb, M_Ai, I_ag], axis=1)
    return _decoder(inp, Wd1, Wd2)
